# baseline (speedup 1.0000x reference)
import functools

import jax
import jax.numpy as jnp
from jax import lax
from jax.experimental import pallas as pl
from jax.experimental.pallas import tpu as pltpu
from jax.experimental.pallas import tpu_sc as plsc

VOCAB = 100000
DIM = 1024
NUM_HEADS = 16
HEAD_DIM = DIM // NUM_HEADS
B = 4
L = 4096

N = B * L
NC = 2
NS = 16
NW = NC * NS
ROWS_PER_W = N // NW
CHUNK = 16
NCHUNK = ROWS_PER_W // CHUNK
NBUF = 6
G = 3


def _gather_body(idx_hbm, table_hbm, out_hbm, idx_v, rows_v, gsem, psem):
  wid = lax.axis_index("s") * NC + lax.axis_index("c")
  base = wid * ROWS_PER_W

  pltpu.sync_copy(idx_hbm.at[wid], idx_v)

  def gather_start(c, buf):
    pltpu.make_async_copy(
        table_hbm.at[idx_v.at[c]], rows_v.at[buf], gsem.at[buf]
    ).start()

  def gather_wait(buf):
    pltpu.make_async_copy(
        table_hbm.at[idx_v.at[0]], rows_v.at[buf], gsem.at[buf]
    ).wait()

  def put_start(c, buf):
    pltpu.make_async_copy(
        rows_v.at[buf], out_hbm.at[pl.ds(base + c * CHUNK, CHUNK)],
        psem.at[buf],
    ).start()

  def put_wait(c, buf):
    pltpu.make_async_copy(
        rows_v.at[buf], out_hbm.at[pl.ds(base + c * CHUNK, CHUNK)],
        psem.at[buf],
    ).wait()

  for j in range(G):
    gather_start(j, j % NBUF)
  for c in range(NCHUNK):
    gather_wait(c % NBUF)
    put_start(c, c % NBUF)
    j = c + G
    if j < NCHUNK:
      if j - NBUF >= 0:
        put_wait(j - NBUF, (j - NBUF) % NBUF)
      gather_start(j, j % NBUF)
  for c in range(max(0, NCHUNK - NBUF), NCHUNK):
    put_wait(c, c % NBUF)


_gather = functools.partial(
    pl.kernel,
    out_type=jax.ShapeDtypeStruct((N, DIM), jnp.float32),
    mesh=plsc.VectorSubcoreMesh(core_axis_name="c", subcore_axis_name="s"),
    scratch_types=[
        pltpu.VMEM((NCHUNK, CHUNK), jnp.int32),
        pltpu.VMEM((NBUF, CHUNK, DIM), jnp.float32),
        pltpu.SemaphoreType.DMA((NBUF,)),
        pltpu.SemaphoreType.DMA((NBUF,)),
    ],
)(_gather_body)


def _rope_body(out_ref):
  half = HEAD_DIM // 2
  t = lax.broadcasted_iota(jnp.int32, (L, half), 0).astype(jnp.float32)
  k = lax.broadcasted_iota(jnp.int32, (L, half), 1).astype(jnp.float32)
  inv_freq = jnp.exp(k * (-2.0 / HEAD_DIM * jnp.log(10000.0)))
  freqs = t * inv_freq
  out_ref[:, :half] = jnp.cos(freqs)
  out_ref[:, half:] = jnp.sin(freqs)


_rope = pl.pallas_call(
    _rope_body,
    out_shape=jax.ShapeDtypeStruct((L, HEAD_DIM), jnp.float32),
)


@jax.jit
def kernel(x, tok_embed):
  idx = x.reshape(NW, NCHUNK, CHUNK)
  h = _gather(idx, tok_embed).reshape(B, L, DIM)
  p = _rope()
  return (h, p)


# speedup vs baseline: 1.0191x; 1.0191x over previous
import functools

import jax
import jax.numpy as jnp
from jax import lax
from jax.experimental import pallas as pl
from jax.experimental.pallas import tpu as pltpu
from jax.experimental.pallas import tpu_sc as plsc

VOCAB = 100000
DIM = 1024
NUM_HEADS = 16
HEAD_DIM = DIM // NUM_HEADS
B = 4
L = 4096

N = B * L
NC = 2
NS = 16
NW = NC * NS
ROWS_PER_W = N // NW
CHUNK = 16
NCHUNK = ROWS_PER_W // CHUNK
NBUF = 7
G = 4


def _gather_body(idx_hbm, table_hbm, out_hbm, idx_v, rows_v, gsem, psem):
  wid = lax.axis_index("s") * NC + lax.axis_index("c")
  base = wid * ROWS_PER_W

  pltpu.sync_copy(idx_hbm.at[wid], idx_v)

  def gather_start(c, buf):
    pltpu.make_async_copy(
        table_hbm.at[idx_v.at[c]], rows_v.at[buf], gsem.at[buf]
    ).start()

  def gather_wait(buf):
    pltpu.make_async_copy(
        table_hbm.at[idx_v.at[0]], rows_v.at[buf], gsem.at[buf]
    ).wait()

  def put_start(c, buf):
    pltpu.make_async_copy(
        rows_v.at[buf], out_hbm.at[pl.ds(base + c * CHUNK, CHUNK)],
        psem.at[buf],
    ).start()

  def put_wait(c, buf):
    pltpu.make_async_copy(
        rows_v.at[buf], out_hbm.at[pl.ds(base + c * CHUNK, CHUNK)],
        psem.at[buf],
    ).wait()

  for j in range(G):
    gather_start(j, j % NBUF)
  for c in range(NCHUNK):
    j = c + G
    if j < NCHUNK:
      if j - NBUF >= 0:
        put_wait(j - NBUF, (j - NBUF) % NBUF)
      gather_start(j, j % NBUF)
    gather_wait(c % NBUF)
    put_start(c, c % NBUF)
  for c in range(max(0, NCHUNK - NBUF), NCHUNK):
    put_wait(c, c % NBUF)


_gather = functools.partial(
    pl.kernel,
    out_type=jax.ShapeDtypeStruct((N, DIM), jnp.float32),
    mesh=plsc.VectorSubcoreMesh(core_axis_name="c", subcore_axis_name="s"),
    scratch_types=[
        pltpu.VMEM((NCHUNK, CHUNK), jnp.int32),
        pltpu.VMEM((NBUF, CHUNK, DIM), jnp.float32),
        pltpu.SemaphoreType.DMA((NBUF,)),
        pltpu.SemaphoreType.DMA((NBUF,)),
    ],
)(_gather_body)


def _rope_body(out_ref):
  half = HEAD_DIM // 2
  t = lax.broadcasted_iota(jnp.int32, (L, half), 0).astype(jnp.float32)
  k = lax.broadcasted_iota(jnp.int32, (L, half), 1).astype(jnp.float32)
  inv_freq = jnp.exp(k * (-2.0 / HEAD_DIM * jnp.log(10000.0)))
  freqs = t * inv_freq
  out_ref[:, :half] = jnp.cos(freqs)
  out_ref[:, half:] = jnp.sin(freqs)


_rope = pl.pallas_call(
    _rope_body,
    out_shape=jax.ShapeDtypeStruct((L, HEAD_DIM), jnp.float32),
)


@jax.jit
def kernel(x, tok_embed):
  idx = x.reshape(NW, NCHUNK, CHUNK)
  h = _gather(idx, tok_embed).reshape(B, L, DIM)
  p = _rope()
  return (h, p)
